# baseline (speedup 1.0000x reference)
import jax
import jax.numpy as jnp
from jax import lax
from jax.experimental import pallas as pl
from jax.experimental.pallas import tpu as pltpu
from jax.experimental.pallas import tpu_sc as plsc

E = 8
D = 1024
H = 4096
T = 2048
M = 256
G = 24
NP = G * M
CH = 16
CB = 128

NC = 2
NS = 16
NW = NC * NS
TPW = T // NW


def _router_body(x_ref, wr_ref, pos_ref, wb1_ref, wb2_ref, meta_ref):
    xv = x_ref[...]
    wr = wr_ref[...]
    logits = lax.dot_general(xv, wr, (((1,), (1,)), ((), ())),
                             preferred_element_type=jnp.float32)

    iota_e = lax.broadcasted_iota(jnp.int32, (T, E), 1)
    m1 = jnp.max(logits, axis=1, keepdims=True)
    a1 = jnp.min(jnp.where(logits == m1, iota_e, E), axis=1)
    masked = jnp.where(iota_e == a1[:, None], -jnp.inf, logits)
    m2 = jnp.max(masked, axis=1, keepdims=True)
    a2 = jnp.min(jnp.where(masked == m2, iota_e, E), axis=1)

    w1 = 1.0 / (1.0 + jnp.exp(m2[:, 0] - m1[:, 0]))
    w2 = 1.0 - w1

    oh1 = (iota_e == a1[:, None]).astype(jnp.float32)
    oh2 = (iota_e == a2[:, None]).astype(jnp.float32)

    ltri = (lax.broadcasted_iota(jnp.int32, (CB, CB), 1)
            < lax.broadcasted_iota(jnp.int32, (CB, CB), 0)).astype(jnp.float32)
    ltri_b = jnp.broadcast_to(ltri[None], (CH, CB, CB))
    ltri_c = (lax.broadcasted_iota(jnp.int32, (CH, CH), 1)
              < lax.broadcasted_iota(jnp.int32, (CH, CH), 0)).astype(jnp.float32)

    def excl_cumsum(oh):
        oh3 = oh.reshape(CH, CB, E)
        sums = jnp.sum(oh3, axis=1)
        offs = lax.dot_general(ltri_c, sums, (((1,), (0,)), ((), ())),
                               preferred_element_type=jnp.float32)
        within = lax.dot_general(ltri_b, oh3, (((2,), (1,)), ((0,), (0,))),
                                 preferred_element_type=jnp.float32)
        return (within + offs[:, None, :]).reshape(T, E)

    se = excl_cumsum(oh1) + excl_cumsum(oh2)
    rank1 = jnp.sum(jnp.where(iota_e == a1[:, None], se, 0.0), axis=1)
    rank2 = jnp.sum(jnp.where(iota_e == a2[:, None], se, 0.0), axis=1)

    cnts = jnp.sum(oh1 + oh2, axis=0, keepdims=True)
    nblk = (cnts.astype(jnp.int32) + (M - 1)) >> 8
    nblk_f = nblk.astype(jnp.float32)
    utri_e = (lax.broadcasted_iota(jnp.int32, (E, E), 0)
              < lax.broadcasted_iota(jnp.int32, (E, E), 1)).astype(jnp.float32)
    poff_blk = lax.dot_general(nblk_f, utri_e, (((1,), (0,)), ((), ())),
                               preferred_element_type=jnp.float32)
    incl_blk = poff_blk + nblk_f
    total_blk = jnp.sum(nblk_f)

    gi = lax.broadcasted_iota(jnp.int32, (G, E), 0).astype(jnp.float32)
    bexp = jnp.sum((gi >= incl_blk).astype(jnp.float32), axis=1)
    iota_e_row = lax.broadcasted_iota(jnp.int32, (1, E), 1).astype(jnp.float32)
    be_last = jnp.max(jnp.where(nblk_f > 0, iota_e_row, 0.0))
    active = gi[:, 0] < total_blk
    bexp = jnp.where(active, bexp, be_last)

    poff_rows = poff_blk * M
    pos1 = rank1 + jnp.sum(
        jnp.where(iota_e == a1[:, None], poff_rows, 0.0), axis=1)
    pos2 = rank2 + jnp.sum(
        jnp.where(iota_e == a2[:, None], poff_rows, 0.0), axis=1)

    startm = jnp.logical_and(gi == poff_blk, nblk_f > 0)
    chg = jnp.logical_and(active, jnp.any(startm, axis=1)).astype(jnp.float32)
    utri_g = (lax.broadcasted_iota(jnp.int32, (G, G), 0)
              <= lax.broadcasted_iota(jnp.int32, (G, G), 1)).astype(jnp.float32)
    inc = lax.dot_general(chg[None, :], utri_g, (((1,), (0,)), ((), ())),
                          preferred_element_type=jnp.float32)[0]
    par = jnp.remainder(inc - 1.0, 2.0)
    runs_total = jnp.sum(chg)
    k_ge = lax.broadcasted_iota(jnp.int32, (G, E), 1).astype(jnp.float32)
    ind = jnp.where(
        jnp.logical_and(inc[:, None] == k_ge + 1.0, chg[:, None] == 1.0),
        bexp[:, None], 0.0)
    runexp0 = jnp.sum(ind, axis=0)
    nxt = jnp.sum(jnp.where(inc[:, None] == k_ge, runexp0[None, :], 0.0),
                  axis=1)
    pf = chg * (inc < runs_total).astype(jnp.float32)

    pos_ref[...] = jnp.concatenate(
        [pos1[None, :], pos2[None, :]], axis=0).astype(jnp.int32)
    wb1_ref[...] = jnp.broadcast_to(w1[:, None], (T, 128))
    wb2_ref[...] = jnp.broadcast_to(w2[:, None], (T, 128))
    meta_ref[...] = jnp.concatenate(
        [active.astype(jnp.float32)[None, :], chg[None, :], par[None, :],
         pf[None, :], nxt[None, :], bexp[None, :],
         jnp.zeros((2, G), jnp.float32)],
        axis=0).astype(jnp.int32)


def _router(x2d, w_router):
    return pl.pallas_call(
        _router_body,
        out_shape=[
            jax.ShapeDtypeStruct((2, T), jnp.int32),
            jax.ShapeDtypeStruct((T, 128), jnp.float32),
            jax.ShapeDtypeStruct((T, 128), jnp.float32),
            jax.ShapeDtypeStruct((8, G), jnp.int32),
        ],
    )(x2d, w_router)


def _dispatch_body(x_hbm, p1_hbm, p2_hbm, wb1_hbm, wb2_hbm, xs_hbm, ws_hbm,
                   rows_v, wrow1_v, wrow2_v, idx1_v, idx2_v, sem):
    wid = lax.axis_index("s") * NC + lax.axis_index("c")
    base = wid * TPW
    pltpu.sync_copy(p1_hbm.at[pl.ds(base, TPW)], idx1_v)
    pltpu.sync_copy(p2_hbm.at[pl.ds(base, TPW)], idx2_v)
    pltpu.sync_copy(x_hbm.at[pl.ds(base, TPW)], rows_v)
    cp1 = pltpu.async_copy(rows_v, xs_hbm.at[idx1_v], sem)
    cp2 = pltpu.async_copy(rows_v, xs_hbm.at[idx2_v], sem)
    pltpu.sync_copy(wb1_hbm.at[pl.ds(base, TPW)], wrow1_v)
    pltpu.sync_copy(wb2_hbm.at[pl.ds(base, TPW)], wrow2_v)
    cp3 = pltpu.async_copy(wrow1_v, ws_hbm.at[idx1_v], sem)
    cp4 = pltpu.async_copy(wrow2_v, ws_hbm.at[idx2_v], sem)
    cp1.wait()
    cp2.wait()
    cp3.wait()
    cp4.wait()


def _dispatch(x2d, p1, p2, wb1, wb2):
    return pl.kernel(
        _dispatch_body,
        out_type=[
            jax.ShapeDtypeStruct((NP, D), jnp.float32),
            jax.ShapeDtypeStruct((NP, 128), jnp.float32),
        ],
        mesh=plsc.VectorSubcoreMesh(core_axis_name="c", subcore_axis_name="s"),
        scratch_types=[
            pltpu.VMEM((TPW, D), jnp.float32),
            pltpu.VMEM((TPW, 128), jnp.float32),
            pltpu.VMEM((TPW, 128), jnp.float32),
            pltpu.VMEM((TPW,), jnp.int32),
            pltpu.VMEM((TPW,), jnp.int32),
            pltpu.SemaphoreType.DMA,
        ],
    )(x2d, p1, p2, wb1, wb2)


HH = H // 2


def _ffn_body(half, m_ref, x_ref, w1_any, w2_any, yin_ref, w_ref, y_ref,
              w1f_ref, w2f_ref, w1b_ref, w2b_ref, sem):
    g = pl.program_id(0)
    bact = m_ref[0, g]
    chg = m_ref[1, g]
    par = m_ref[2, g]
    pf = m_ref[3, g]
    nxt = m_ref[4, g]
    bee = m_ref[5, g]

    def copies(e, buf):
        c1 = pltpu.make_async_copy(
            w1_any.at[e, pl.ds(half * HH, HH), :], w1f_ref.at[buf], sem)
        c2 = pltpu.make_async_copy(
            w2_any.at[e, :, pl.ds(half * HH, HH)], w2f_ref.at[buf], sem)
        return c1, c2

    @pl.when(g == 0)
    def _():
        c1, c2 = copies(bee, 0)
        c1.start()
        c2.start()
        c1.wait()
        c2.wait()

    @pl.when(jnp.logical_and(g > 0, chg == 1))
    def _():
        c1, c2 = copies(0, 0)
        c1.wait()
        c2.wait()

    def consume(buf):
        w1b_ref[...] = w1f_ref[buf].astype(jnp.bfloat16)
        w2b_ref[...] = w2f_ref[buf].astype(jnp.bfloat16)

        @pl.when(pf == 1)
        def _():
            c1, c2 = copies(nxt, 1 - buf)
            c1.start()
            c2.start()

    @pl.when(jnp.logical_and(chg == 1, par == 0))
    def _():
        consume(0)

    @pl.when(jnp.logical_and(chg == 1, par == 1))
    def _():
        consume(1)

    @pl.when(bact == 1)
    def _():
        xb = x_ref[...].astype(jnp.bfloat16)
        h = lax.dot_general(xb, w1b_ref[...], (((1,), (1,)), ((), ())),
                            preferred_element_type=jnp.float32)
        h = jnp.maximum(h, 0.0)
        h = (h * h).astype(jnp.bfloat16)
        y = lax.dot_general(h, w2b_ref[...], (((1,), (1,)), ((), ())),
                            preferred_element_type=jnp.float32)
        if half == 1:
            y_ref[...] = (y + yin_ref[...].astype(jnp.float32)) * w_ref[...][:, 0:1]
        else:
            y_ref[...] = y.astype(jnp.bfloat16)


def _ffn_half(meta, x_sorted, fc1_w, fc2_w, y_prev, w_sorted, half):
    in_specs = [
        pl.BlockSpec((M, D), lambda g, m: (g, 0)),
        pl.BlockSpec(memory_space=pl.ANY),
        pl.BlockSpec(memory_space=pl.ANY),
    ]
    args = [meta, x_sorted, fc1_w, fc2_w]
    if half == 1:
        in_specs.append(pl.BlockSpec((M, D), lambda g, m: (g, 0)))
        in_specs.append(pl.BlockSpec((M, 128), lambda g, m: (g, 0)))
        args.append(y_prev)
        args.append(w_sorted)
    if half == 0:
        body = (lambda m, x, w1, w2, y, s1, s2, s3, s4, sem:
                _ffn_body(0, m, x, w1, w2, None, None, y, s1, s2, s3, s4, sem))
    else:
        body = (lambda m, x, w1, w2, yin, w, y, s1, s2, s3, s4, sem:
                _ffn_body(1, m, x, w1, w2, yin, w, y, s1, s2, s3, s4, sem))
    grid_spec = pltpu.PrefetchScalarGridSpec(
        num_scalar_prefetch=1,
        grid=(G,),
        in_specs=in_specs,
        out_specs=pl.BlockSpec((M, D), lambda g, m: (g, 0)),
        scratch_shapes=[
            pltpu.VMEM((2, HH, D), jnp.float32),
            pltpu.VMEM((2, D, HH), jnp.float32),
            pltpu.VMEM((HH, D), jnp.bfloat16),
            pltpu.VMEM((D, HH), jnp.bfloat16),
            pltpu.SemaphoreType.DMA,
        ],
    )
    out_dt = jnp.bfloat16 if half == 0 else jnp.float32
    return pl.pallas_call(
        body,
        grid_spec=grid_spec,
        out_shape=jax.ShapeDtypeStruct((NP, D), out_dt),
    )(*args)


def _ffn(meta, x_sorted, fc1_w, fc2_w, w_sorted):
    y_a = _ffn_half(meta, x_sorted, fc1_w, fc2_w, None, None, 0)
    return _ffn_half(meta, x_sorted, fc1_w, fc2_w, y_a, w_sorted, 1)


def _combine_body(y_hbm, p1_hbm, p2_hbm, out_hbm,
                  rows1_v, rows2_v, outb_v, idx1_v, idx2_v, sem):
    wid = lax.axis_index("s") * NC + lax.axis_index("c")
    for c in range(TPW // 16):
        base = wid * TPW + c * 16
        pltpu.sync_copy(p1_hbm.at[pl.ds(base, 16)], idx1_v)
        pltpu.sync_copy(p2_hbm.at[pl.ds(base, 16)], idx2_v)
        cp1 = pltpu.async_copy(y_hbm.at[idx1_v], rows1_v, sem)
        cp2 = pltpu.async_copy(y_hbm.at[idx2_v], rows2_v, sem)
        cp1.wait()
        cp2.wait()

        def tok(t, carry):
            for j in range(D // 16):
                sl = pl.ds(j * 16, 16)
                outb_v[t, sl] = rows1_v[t, sl] + rows2_v[t, sl]
            return carry

        lax.fori_loop(0, 16, tok, 0)
        pltpu.sync_copy(outb_v, out_hbm.at[pl.ds(base, 16)])


def _combine(y_sorted, p1, p2):
    return pl.kernel(
        _combine_body,
        out_type=jax.ShapeDtypeStruct((T, D), jnp.float32),
        mesh=plsc.VectorSubcoreMesh(core_axis_name="c", subcore_axis_name="s"),
        scratch_types=[
            pltpu.VMEM((16, D), jnp.float32),
            pltpu.VMEM((16, D), jnp.float32),
            pltpu.VMEM((16, D), jnp.float32),
            pltpu.VMEM((16,), jnp.int32),
            pltpu.VMEM((16,), jnp.int32),
            pltpu.SemaphoreType.DMA,
        ],
    )(y_sorted, p1, p2)


@jax.jit
def kernel(x, W_router, fc1_w, fc2_w):
    x2d = x.reshape(T, D)
    pos, wb1, wb2, meta = _router(x2d, W_router)
    p1, p2 = pos[0], pos[1]
    x_sorted, w_sorted = _dispatch(x2d, p1, p2, wb1, wb2)
    y_sorted = _ffn(meta, x_sorted, fc1_w, fc2_w, w_sorted)
    out2d = _combine(y_sorted, p1, p2)
    return out2d.reshape(1, T, D)


# speedup vs baseline: 1.0180x; 1.0180x over previous
import jax
import jax.numpy as jnp
from jax import lax
from jax.experimental import pallas as pl
from jax.experimental.pallas import tpu as pltpu
from jax.experimental.pallas import tpu_sc as plsc

E = 8
D = 1024
H = 4096
T = 2048
M = 256
G = 24
NP = G * M
CH = 16
CB = 128

NC = 2
NS = 16
NW = NC * NS
TPW = T // NW


def _router_body(x_ref, wr_ref, pos_ref, wb1_ref, wb2_ref, meta_ref):
    xv = x_ref[...]
    wr = wr_ref[...]
    logits = lax.dot_general(xv, wr, (((1,), (1,)), ((), ())),
                             preferred_element_type=jnp.float32)

    iota_e = lax.broadcasted_iota(jnp.int32, (T, E), 1)
    m1 = jnp.max(logits, axis=1, keepdims=True)
    a1 = jnp.min(jnp.where(logits == m1, iota_e, E), axis=1)
    masked = jnp.where(iota_e == a1[:, None], -jnp.inf, logits)
    m2 = jnp.max(masked, axis=1, keepdims=True)
    a2 = jnp.min(jnp.where(masked == m2, iota_e, E), axis=1)

    w1 = 1.0 / (1.0 + jnp.exp(m2[:, 0] - m1[:, 0]))
    w2 = 1.0 - w1

    oh1 = (iota_e == a1[:, None]).astype(jnp.float32)
    oh2 = (iota_e == a2[:, None]).astype(jnp.float32)

    ltri = (lax.broadcasted_iota(jnp.int32, (CB, CB), 1)
            < lax.broadcasted_iota(jnp.int32, (CB, CB), 0)).astype(jnp.float32)
    ltri_b = jnp.broadcast_to(ltri[None], (CH, CB, CB))
    ltri_c = (lax.broadcasted_iota(jnp.int32, (CH, CH), 1)
              < lax.broadcasted_iota(jnp.int32, (CH, CH), 0)).astype(jnp.float32)

    def excl_cumsum(oh):
        oh3 = oh.reshape(CH, CB, E)
        sums = jnp.sum(oh3, axis=1)
        offs = lax.dot_general(ltri_c, sums, (((1,), (0,)), ((), ())),
                               preferred_element_type=jnp.float32)
        within = lax.dot_general(ltri_b, oh3, (((2,), (1,)), ((0,), (0,))),
                                 preferred_element_type=jnp.float32)
        return (within + offs[:, None, :]).reshape(T, E)

    se = excl_cumsum(oh1) + excl_cumsum(oh2)
    rank1 = jnp.sum(jnp.where(iota_e == a1[:, None], se, 0.0), axis=1)
    rank2 = jnp.sum(jnp.where(iota_e == a2[:, None], se, 0.0), axis=1)

    cnts = jnp.sum(oh1 + oh2, axis=0, keepdims=True)
    nblk = (cnts.astype(jnp.int32) + (M - 1)) >> 8
    nblk_f = nblk.astype(jnp.float32)
    utri_e = (lax.broadcasted_iota(jnp.int32, (E, E), 0)
              < lax.broadcasted_iota(jnp.int32, (E, E), 1)).astype(jnp.float32)
    poff_blk = lax.dot_general(nblk_f, utri_e, (((1,), (0,)), ((), ())),
                               preferred_element_type=jnp.float32)
    incl_blk = poff_blk + nblk_f
    total_blk = jnp.sum(nblk_f)

    gi = lax.broadcasted_iota(jnp.int32, (G, E), 0).astype(jnp.float32)
    bexp = jnp.sum((gi >= incl_blk).astype(jnp.float32), axis=1)
    iota_e_row = lax.broadcasted_iota(jnp.int32, (1, E), 1).astype(jnp.float32)
    be_last = jnp.max(jnp.where(nblk_f > 0, iota_e_row, 0.0))
    active = gi[:, 0] < total_blk
    bexp = jnp.where(active, bexp, be_last)

    poff_rows = poff_blk * M
    pos1 = rank1 + jnp.sum(
        jnp.where(iota_e == a1[:, None], poff_rows, 0.0), axis=1)
    pos2 = rank2 + jnp.sum(
        jnp.where(iota_e == a2[:, None], poff_rows, 0.0), axis=1)

    startm = jnp.logical_and(gi == poff_blk, nblk_f > 0)
    chg = jnp.logical_and(active, jnp.any(startm, axis=1)).astype(jnp.float32)
    utri_g = (lax.broadcasted_iota(jnp.int32, (G, G), 0)
              <= lax.broadcasted_iota(jnp.int32, (G, G), 1)).astype(jnp.float32)
    inc = lax.dot_general(chg[None, :], utri_g, (((1,), (0,)), ((), ())),
                          preferred_element_type=jnp.float32)[0]
    par = jnp.remainder(inc - 1.0, 2.0)
    runs_total = jnp.sum(chg)
    k_ge = lax.broadcasted_iota(jnp.int32, (G, E), 1).astype(jnp.float32)
    ind = jnp.where(
        jnp.logical_and(inc[:, None] == k_ge + 1.0, chg[:, None] == 1.0),
        bexp[:, None], 0.0)
    runexp0 = jnp.sum(ind, axis=0)
    nxt = jnp.sum(jnp.where(inc[:, None] == k_ge, runexp0[None, :], 0.0),
                  axis=1)
    pf = chg * (inc < runs_total).astype(jnp.float32)

    pos_ref[...] = jnp.concatenate(
        [pos1[None, :], pos2[None, :]], axis=0).astype(jnp.int32)
    wb1_ref[...] = jnp.broadcast_to(w1[:, None], (T, 16))
    wb2_ref[...] = jnp.broadcast_to(w2[:, None], (T, 16))
    meta_ref[...] = jnp.concatenate(
        [active.astype(jnp.float32)[None, :], chg[None, :], par[None, :],
         pf[None, :], nxt[None, :], bexp[None, :],
         jnp.zeros((2, G), jnp.float32)],
        axis=0).astype(jnp.int32)


def _router(x2d, w_router):
    return pl.pallas_call(
        _router_body,
        out_shape=[
            jax.ShapeDtypeStruct((2, T), jnp.int32),
            jax.ShapeDtypeStruct((T, 16), jnp.float32),
            jax.ShapeDtypeStruct((T, 16), jnp.float32),
            jax.ShapeDtypeStruct((8, G), jnp.int32),
        ],
    )(x2d, w_router)


def _dispatch_body(x_hbm, p1_hbm, p2_hbm, xs_hbm,
                   rows_v, idx1_v, idx2_v, sem):
    wid = lax.axis_index("s") * NC + lax.axis_index("c")
    base = wid * TPW
    pltpu.sync_copy(p1_hbm.at[pl.ds(base, TPW)], idx1_v)
    pltpu.sync_copy(p2_hbm.at[pl.ds(base, TPW)], idx2_v)
    pltpu.sync_copy(x_hbm.at[pl.ds(base, TPW)], rows_v)
    cp1 = pltpu.async_copy(rows_v, xs_hbm.at[idx1_v], sem)
    cp2 = pltpu.async_copy(rows_v, xs_hbm.at[idx2_v], sem)
    cp1.wait()
    cp2.wait()


def _dispatch(x2d, p1, p2):
    return pl.kernel(
        _dispatch_body,
        out_type=jax.ShapeDtypeStruct((NP, D), jnp.float32),
        mesh=plsc.VectorSubcoreMesh(core_axis_name="c", subcore_axis_name="s"),
        scratch_types=[
            pltpu.VMEM((TPW, D), jnp.float32),
            pltpu.VMEM((TPW,), jnp.int32),
            pltpu.VMEM((TPW,), jnp.int32),
            pltpu.SemaphoreType.DMA,
        ],
    )(x2d, p1, p2)


HH = H // 2


def _ffn_body(half, m_ref, x_ref, w1_any, w2_any, yin_ref, y_ref,
              w1f_ref, w2f_ref, w1b_ref, w2b_ref, sem):
    g = pl.program_id(0)
    bact = m_ref[0, g]
    chg = m_ref[1, g]
    par = m_ref[2, g]
    pf = m_ref[3, g]
    nxt = m_ref[4, g]
    bee = m_ref[5, g]

    def copies(e, buf):
        c1 = pltpu.make_async_copy(
            w1_any.at[e, pl.ds(half * HH, HH), :], w1f_ref.at[buf], sem)
        c2 = pltpu.make_async_copy(
            w2_any.at[e, :, pl.ds(half * HH, HH)], w2f_ref.at[buf], sem)
        return c1, c2

    @pl.when(g == 0)
    def _():
        c1, c2 = copies(bee, 0)
        c1.start()
        c2.start()
        c1.wait()
        c2.wait()

    @pl.when(jnp.logical_and(g > 0, chg == 1))
    def _():
        c1, c2 = copies(0, 0)
        c1.wait()
        c2.wait()

    def consume(buf):
        w1b_ref[...] = w1f_ref[buf].astype(jnp.bfloat16)
        w2b_ref[...] = w2f_ref[buf].astype(jnp.bfloat16)

        @pl.when(pf == 1)
        def _():
            c1, c2 = copies(nxt, 1 - buf)
            c1.start()
            c2.start()

    @pl.when(jnp.logical_and(chg == 1, par == 0))
    def _():
        consume(0)

    @pl.when(jnp.logical_and(chg == 1, par == 1))
    def _():
        consume(1)

    @pl.when(bact == 1)
    def _():
        xb = x_ref[...].astype(jnp.bfloat16)
        h = lax.dot_general(xb, w1b_ref[...], (((1,), (1,)), ((), ())),
                            preferred_element_type=jnp.float32)
        h = jnp.maximum(h, 0.0)
        h = (h * h).astype(jnp.bfloat16)
        y = lax.dot_general(h, w2b_ref[...], (((1,), (1,)), ((), ())),
                            preferred_element_type=jnp.float32)
        if half == 1:
            y_ref[...] = y + yin_ref[...].astype(jnp.float32)
        else:
            y_ref[...] = y.astype(jnp.bfloat16)


def _ffn_half(meta, x_sorted, fc1_w, fc2_w, y_prev, half):
    in_specs = [
        pl.BlockSpec((M, D), lambda g, m: (g, 0)),
        pl.BlockSpec(memory_space=pl.ANY),
        pl.BlockSpec(memory_space=pl.ANY),
    ]
    args = [meta, x_sorted, fc1_w, fc2_w]
    if half == 1:
        in_specs.append(pl.BlockSpec((M, D), lambda g, m: (g, 0)))
        args.append(y_prev)
    if half == 0:
        body = (lambda m, x, w1, w2, y, s1, s2, s3, s4, sem:
                _ffn_body(0, m, x, w1, w2, None, y, s1, s2, s3, s4, sem))
    else:
        body = (lambda m, x, w1, w2, yin, y, s1, s2, s3, s4, sem:
                _ffn_body(1, m, x, w1, w2, yin, y, s1, s2, s3, s4, sem))
    grid_spec = pltpu.PrefetchScalarGridSpec(
        num_scalar_prefetch=1,
        grid=(G,),
        in_specs=in_specs,
        out_specs=pl.BlockSpec((M, D), lambda g, m: (g, 0)),
        scratch_shapes=[
            pltpu.VMEM((2, HH, D), jnp.float32),
            pltpu.VMEM((2, D, HH), jnp.float32),
            pltpu.VMEM((HH, D), jnp.bfloat16),
            pltpu.VMEM((D, HH), jnp.bfloat16),
            pltpu.SemaphoreType.DMA,
        ],
    )
    out_dt = jnp.bfloat16 if half == 0 else jnp.float32
    return pl.pallas_call(
        body,
        grid_spec=grid_spec,
        out_shape=jax.ShapeDtypeStruct((NP, D), out_dt),
    )(*args)


def _ffn(meta, x_sorted, fc1_w, fc2_w):
    y_a = _ffn_half(meta, x_sorted, fc1_w, fc2_w, None, 0)
    return _ffn_half(meta, x_sorted, fc1_w, fc2_w, y_a, 1)


def _combine_body(y_hbm, p1_hbm, p2_hbm, wb1_hbm, wb2_hbm, out_hbm,
                  rows1_v, rows2_v, outb_v, wb1_v, wb2_v, idx1_v, idx2_v, sem):
    wid = lax.axis_index("s") * NC + lax.axis_index("c")
    for c in range(TPW // 16):
        base = wid * TPW + c * 16
        pltpu.sync_copy(p1_hbm.at[pl.ds(base, 16)], idx1_v)
        pltpu.sync_copy(p2_hbm.at[pl.ds(base, 16)], idx2_v)
        cp1 = pltpu.async_copy(y_hbm.at[idx1_v], rows1_v, sem)
        cp2 = pltpu.async_copy(y_hbm.at[idx2_v], rows2_v, sem)
        pltpu.sync_copy(wb1_hbm.at[pl.ds(base, 16)], wb1_v)
        pltpu.sync_copy(wb2_hbm.at[pl.ds(base, 16)], wb2_v)
        cp1.wait()
        cp2.wait()

        def tok(t, carry):
            w1r = wb1_v[t]
            w2r = wb2_v[t]
            for j in range(D // 16):
                sl = pl.ds(j * 16, 16)
                outb_v[t, sl] = w1r * rows1_v[t, sl] + w2r * rows2_v[t, sl]
            return carry

        lax.fori_loop(0, 16, tok, 0)
        pltpu.sync_copy(outb_v, out_hbm.at[pl.ds(base, 16)])


def _combine(y_sorted, p1, p2, wb1, wb2):
    return pl.kernel(
        _combine_body,
        out_type=jax.ShapeDtypeStruct((T, D), jnp.float32),
        mesh=plsc.VectorSubcoreMesh(core_axis_name="c", subcore_axis_name="s"),
        scratch_types=[
            pltpu.VMEM((16, D), jnp.float32),
            pltpu.VMEM((16, D), jnp.float32),
            pltpu.VMEM((16, D), jnp.float32),
            pltpu.VMEM((16, 16), jnp.float32),
            pltpu.VMEM((16, 16), jnp.float32),
            pltpu.VMEM((16,), jnp.int32),
            pltpu.VMEM((16,), jnp.int32),
            pltpu.SemaphoreType.DMA,
        ],
    )(y_sorted, p1, p2, wb1, wb2)


@jax.jit
def kernel(x, W_router, fc1_w, fc2_w):
    x2d = x.reshape(T, D)
    pos, wb1, wb2, meta = _router(x2d, W_router)
    p1, p2 = pos[0], pos[1]
    x_sorted = _dispatch(x2d, p1, p2)
    y_sorted = _ffn(meta, x_sorted, fc1_w, fc2_w)
    out2d = _combine(y_sorted, p1, p2, wb1, wb2)
    return out2d.reshape(1, T, D)
